# baseline (speedup 1.0000x reference)
import functools

import jax
import jax.numpy as jnp
from jax.experimental import pallas as pl
from jax.experimental.pallas import tpu as pltpu

T = 0.1
EPS_BN = 1e-5
CC_ITERS = 32
INF = 1e9


def _dot_t(a, b):
    return jax.lax.dot_general(a, b, (((1,), (1,)), ((), ())),
                               preferred_element_type=jnp.float32)


def _mm1_kernel(x1_ref, x2_ref, w1_ref, b1_ref, h_ref, s_ref, q_ref):
    hd = pl.program_id(0)
    m = pl.program_id(1)
    xb = jnp.where(hd == 0, x1_ref[...], x2_ref[...]).astype(jnp.bfloat16)
    hf = _dot_t(xb, w1_ref[...]) + b1_ref[...]
    h_ref[...] = hf.astype(jnp.bfloat16)
    ps = jnp.sum(hf, axis=0, keepdims=True)[None]
    pq = jnp.sum(hf * hf, axis=0, keepdims=True)[None]

    @pl.when(m == 0)
    def _():
        s_ref[...] = ps
        q_ref[...] = pq

    @pl.when(m > 0)
    def _():
        s_ref[...] = s_ref[...] + ps
        q_ref[...] = q_ref[...] + pq


def _mm2_kernel(h_ref, s_ref, q_ref, g_ref, be_ref, w2_ref, b2_ref,
                f_ref, *, batch):
    mu = s_ref[0] / batch
    var = q_ref[0] / batch - mu * mu
    scale = g_ref[...] * jax.lax.rsqrt(var + EPS_BN)
    shift = be_ref[...] - mu * scale
    hn = jnp.maximum(h_ref[...].astype(jnp.float32) * scale + shift, 0.0)
    of = _dot_t(hn.astype(jnp.bfloat16), w2_ref[...]) + b2_ref[...]
    ss = jnp.sum(of * of, axis=1, keepdims=True)
    inv = jax.lax.rsqrt(jnp.maximum(ss, 1e-24))
    f_ref[...] = (of * inv).astype(jnp.bfloat16)


def _gram_kernel(fb_ref, ff_ref, lg_ref, y_ref, dl_ref, *, bm, b):
    gf = _dot_t(fb_ref[...], ff_ref[...])
    lgf = gf * (1.0 / T)
    lg_ref[...] = lgf.astype(jnp.bfloat16)
    m = pl.program_id(1)
    riota = jax.lax.broadcasted_iota(jnp.int32, (bm, b), 0) + m * bm
    ciota = jax.lax.broadcasted_iota(jnp.int32, (bm, b), 1)
    offd = riota != ciota
    expl = jnp.where(offd, jnp.exp(lgf), 0.0)
    dlog = jnp.log(jnp.sum(expl, axis=1, keepdims=True))
    dl_ref[...] = jnp.broadcast_to(dlog, (bm, 128))
    sim = gf - jnp.where(riota == ciota, 2.0, 0.0)
    smax = jnp.max(sim, axis=1, keepdims=True)
    y = jnp.min(jnp.where(sim == smax, ciota, b), axis=1, keepdims=True)
    y_ref[...] = jnp.broadcast_to(y, (bm, 128))


def _row_to_col(row8, b):
    slabs = []
    for a in range(b // 128):
        sl = row8[:, 128 * a:128 * (a + 1)]
        rep = jnp.tile(sl, (16, 1))
        slabs.append(jnp.transpose(rep))
    return jnp.concatenate(slabs, axis=0)


def _col_to_row8(col, b):
    outs = []
    for a in range(b // 128):
        t = jnp.transpose(col[128 * a:128 * (a + 1), :])
        outs.append(t[0:8, :])
    return jnp.concatenate(outs, axis=1)


def _min_rows(get_chunk, b):
    acc = jnp.full((8, b), INF, jnp.float32)
    for r0 in range(0, b, 64):
        blk = get_chunk(r0)
        for k in range(8):
            acc = jnp.minimum(acc, blk[8 * k:8 * (k + 1), :])
    return jnp.min(acc, axis=0, keepdims=True)


def _v16_to_col(v16, b):
    slabs = []
    for a in range(b // 128):
        rep = jnp.broadcast_to(v16[a:a + 1, :], (128, 128))
        slabs.append(jnp.transpose(rep))
    return jnp.concatenate(slabs, axis=0)


def _row_to_v16(row, b):
    return jnp.concatenate(
        [row[0:1, 128 * a:128 * (a + 1)] for a in range(b // 128)], axis=0)


def _gather16(v16, r, c, b):
    out = jnp.full((b // 128, 128), INF, jnp.float32)
    for a in range(b // 128):
        rowb = jnp.broadcast_to(v16[a:a + 1, :], (b // 128, 128))
        g_a = jnp.take_along_axis(rowb, c, axis=1)
        out = jnp.where(r == a, g_a, out)
    return out


def _cc_kernel(y_ref, lr_ref, lc_ref, yc_ref, *, b):
    nrep = b // 128
    ycol = y_ref[...]
    ciota = jax.lax.broadcasted_iota(jnp.int32, (b, b), 1)
    yc_ref[...] = jnp.where(jnp.tile(ycol, (1, nrep)) == ciota, 0.0, INF)
    y16 = _row_to_v16(_col_to_row8(ycol, b)[0:1, :], b)
    ry = jax.lax.shift_right_logical(y16, 7)
    cy = jnp.bitwise_and(y16, 127)

    l16 = (jax.lax.broadcasted_iota(jnp.int32, (nrep, 128), 0) * 128 +
           jax.lax.broadcasted_iota(jnp.int32, (nrep, 128), 1)
           ).astype(jnp.float32)

    def body(carry):
        it, _, l16 = carry
        g = _gather16(l16, ry, cy, b)
        ln = jnp.minimum(l16, g)
        l_col = _v16_to_col(l16, b)

        def chunk(r0):
            cl = jnp.tile(l_col[r0:r0 + 64, :], (1, nrep))
            return yc_ref[r0:r0 + 64, :] + cl

        s16 = _row_to_v16(_min_rows(chunk, b), b)
        ln2 = jnp.minimum(ln, s16)
        ln2i = ln2.astype(jnp.int32)
        c16 = _gather16(ln2, jax.lax.shift_right_logical(ln2i, 7),
                        jnp.bitwise_and(ln2i, 127), b)
        l_new = jnp.minimum(ln2, c16)
        changed = jnp.any(l_new != l16)
        return it + 1, changed, l_new

    def cond(carry):
        it, changed, _ = carry
        return jnp.logical_and(it < CC_ITERS, changed)

    _, _, l16 = jax.lax.while_loop(
        cond, body, (jnp.int32(0), jnp.bool_(True), l16))
    row = jnp.concatenate(
        [l16[a:a + 1, :] for a in range(nrep)], axis=1)
    lr_ref[...] = jnp.broadcast_to(row, (8, b)).astype(jnp.int32)[None]
    lc_ref[...] = _v16_to_col(l16, b).astype(jnp.int32)


def _loss_kernel(lg_ref, lc_ref, lr_ref, dl_ref, acc_ref, *, bm, b):
    m = pl.program_id(1)
    lg = lg_ref[...].astype(jnp.float32)
    lab_c = jnp.tile(lc_ref[...], (1, b // 128))
    lab_r = jnp.tile(lr_ref[0], (bm // 8, 1))
    riota = jax.lax.broadcasted_iota(jnp.int32, (bm, b), 0) + m * bm
    ciota = jax.lax.broadcasted_iota(jnp.int32, (bm, b), 1)
    offd = riota != ciota
    mask = jnp.where(jnp.logical_and(lab_c == lab_r, offd), 1.0, 0.0)
    s = jnp.sum(mask * lg, axis=1, keepdims=True)
    cnt = jnp.sum(mask, axis=1, keepdims=True)
    terms = s / cnt - dl_ref[...]
    part = jnp.sum(terms, axis=0, keepdims=True)[None]

    @pl.when(m == 0)
    def _():
        acc_ref[...] = jnp.zeros_like(acc_ref)

    acc_ref[...] = acc_ref[...] + jnp.broadcast_to(part, acc_ref.shape)


def kernel(x1, x2, W1, b1, gamma, beta, W2, b2):
    b, d = x1.shape
    h = W1.shape[0]
    bm = min(512, b)
    mb = b // bm
    bm2 = min(512, b)
    mb2 = b // bm2
    f32 = jnp.float32
    params = pltpu.CompilerParams(
        dimension_semantics=("parallel", "arbitrary"),
        vmem_limit_bytes=56 * 1024 * 1024)

    w1b = W1.astype(jnp.bfloat16)
    w2b = W2.astype(jnp.bfloat16)
    b1r = b1.reshape(1, h)
    gr = gamma.reshape(1, h)
    ber = beta.reshape(1, h)
    b2r = b2.reshape(1, d)

    hs, sums, sumsq = pl.pallas_call(
        _mm1_kernel,
        grid=(2, mb),
        in_specs=[
            pl.BlockSpec((bm, d), lambda hd, m: (m * (1 - hd), 0)),
            pl.BlockSpec((bm, d), lambda hd, m: (m * hd, 0)),
            pl.BlockSpec((h, d), lambda hd, m: (0, 0)),
            pl.BlockSpec((1, h), lambda hd, m: (0, 0)),
        ],
        out_specs=[
            pl.BlockSpec((bm, h), lambda hd, m: (hd * mb + m, 0)),
            pl.BlockSpec((1, 1, h), lambda hd, m: (hd, 0, 0)),
            pl.BlockSpec((1, 1, h), lambda hd, m: (hd, 0, 0)),
        ],
        out_shape=[
            jax.ShapeDtypeStruct((2 * b, h), jnp.bfloat16),
            jax.ShapeDtypeStruct((2, 1, h), f32),
            jax.ShapeDtypeStruct((2, 1, h), f32),
        ],
        compiler_params=params,
        name="wcl_mm1",
    )(x1, x2, w1b, b1r)

    feat = pl.pallas_call(
        functools.partial(_mm2_kernel, batch=float(b)),
        grid=(2, mb2),
        in_specs=[
            pl.BlockSpec((bm2, h), lambda hd, m: (hd * mb2 + m, 0)),
            pl.BlockSpec((1, 1, h), lambda hd, m: (hd, 0, 0)),
            pl.BlockSpec((1, 1, h), lambda hd, m: (hd, 0, 0)),
            pl.BlockSpec((1, h), lambda hd, m: (0, 0)),
            pl.BlockSpec((1, h), lambda hd, m: (0, 0)),
            pl.BlockSpec((d, h), lambda hd, m: (0, 0)),
            pl.BlockSpec((1, d), lambda hd, m: (0, 0)),
        ],
        out_specs=pl.BlockSpec((bm2, d), lambda hd, m: (hd * mb2 + m, 0)),
        out_shape=jax.ShapeDtypeStruct((2 * b, d), jnp.bfloat16),
        compiler_params=params,
        name="wcl_mm2",
    )(hs, sums, sumsq, gr, ber, w2b, b2r)

    logits, ycol, dlog = pl.pallas_call(
        functools.partial(_gram_kernel, bm=bm2, b=b),
        grid=(2, mb2),
        in_specs=[
            pl.BlockSpec((bm2, d), lambda hd, m: (hd * mb2 + m, 0)),
            pl.BlockSpec((b, d), lambda hd, m: (hd, 0)),
        ],
        out_specs=[
            pl.BlockSpec((bm2, b), lambda hd, m: (hd * mb2 + m, 0)),
            pl.BlockSpec((bm2, 128), lambda hd, m: (hd * mb2 + m, 0)),
            pl.BlockSpec((bm2, 128), lambda hd, m: (hd * mb2 + m, 0)),
        ],
        out_shape=[
            jax.ShapeDtypeStruct((2 * b, b), jnp.bfloat16),
            jax.ShapeDtypeStruct((2 * b, 128), jnp.int32),
            jax.ShapeDtypeStruct((2 * b, 128), f32),
        ],
        compiler_params=params,
        name="wcl_gram",
    )(feat, feat)

    labrow, labcol = pl.pallas_call(
        functools.partial(_cc_kernel, b=b),
        grid=(2,),
        in_specs=[pl.BlockSpec((b, 128), lambda hd: (hd, 0))],
        out_specs=[
            pl.BlockSpec((1, 8, b), lambda hd: (hd, 0, 0)),
            pl.BlockSpec((b, 128), lambda hd: (hd, 0)),
        ],
        out_shape=[
            jax.ShapeDtypeStruct((2, 8, b), jnp.int32),
            jax.ShapeDtypeStruct((2 * b, 128), jnp.int32),
        ],
        scratch_shapes=[
            pltpu.VMEM((b, b), f32),
        ],
        compiler_params=pltpu.CompilerParams(
            dimension_semantics=("parallel",),
            vmem_limit_bytes=56 * 1024 * 1024),
        name="wcl_cc",
    )(ycol)

    acc = pl.pallas_call(
        functools.partial(_loss_kernel, bm=bm2, b=b),
        grid=(2, mb2),
        in_specs=[
            pl.BlockSpec((bm2, b), lambda hd, m: (hd * mb2 + m, 0)),
            pl.BlockSpec((bm2, 128), lambda hd, m: ((1 - hd) * mb2 + m, 0)),
            pl.BlockSpec((1, 8, b), lambda hd, m: (1 - hd, 0, 0)),
            pl.BlockSpec((bm2, 128), lambda hd, m: (hd * mb2 + m, 0)),
        ],
        out_specs=pl.BlockSpec((1, 8, 128), lambda hd, m: (hd, 0, 0)),
        out_shape=jax.ShapeDtypeStruct((2, 8, 128), f32),
        compiler_params=params,
        name="wcl_loss",
    )(logits, labcol, labrow, dlog)

    return -(acc[0, 0, 0] + acc[1, 0, 0]) / (2.0 * b)


# speedup vs baseline: 1.0407x; 1.0407x over previous
import functools

import jax
import jax.numpy as jnp
from jax.experimental import pallas as pl
from jax.experimental.pallas import tpu as pltpu

T = 0.1
EPS_BN = 1e-5
CC_ITERS = 32
INF = 1e9


def _dot_t(a, b):
    return jax.lax.dot_general(a, b, (((1,), (1,)), ((), ())),
                               preferred_element_type=jnp.float32)


def _head_gram_kernel(x1_ref, x2_ref, w_ref, b1_ref, g_ref, be_ref, b2_ref,
                      lg_ref, y_ref, dl_ref, hf_sc, s_sc, q_sc,
                      *, bm, b, batch):
    hd = pl.program_id(0)
    ph = pl.program_id(1)
    m = pl.program_id(2)
    rows = pl.ds(pl.multiple_of(m * bm, bm), bm)

    @pl.when(ph == 0)
    def _():
        xb = jnp.where(hd == 0, x1_ref[...], x2_ref[...]).astype(jnp.bfloat16)
        hf = _dot_t(xb, w_ref[0]) + b1_ref[...]
        hf_sc[rows, :] = hf.astype(jnp.bfloat16)
        ps = jnp.sum(hf, axis=0, keepdims=True)
        pq = jnp.sum(hf * hf, axis=0, keepdims=True)

        @pl.when(m == 0)
        def _():
            s_sc[...] = ps
            q_sc[...] = pq

        @pl.when(m > 0)
        def _():
            s_sc[...] = s_sc[...] + ps
            q_sc[...] = q_sc[...] + pq

    @pl.when(ph == 1)
    def _():
        mu = s_sc[...] / batch
        var = q_sc[...] / batch - mu * mu
        scale = g_ref[...] * jax.lax.rsqrt(var + EPS_BN)
        shift = be_ref[...] - mu * scale
        hn = jnp.maximum(
            hf_sc[rows, :].astype(jnp.float32) * scale + shift, 0.0)
        of = _dot_t(hn.astype(jnp.bfloat16), w_ref[0]) + b2_ref[...]
        ss = jnp.sum(of * of, axis=1, keepdims=True)
        inv = jax.lax.rsqrt(jnp.maximum(ss, 1e-24))
        hf_sc[rows, :] = (of * inv).astype(jnp.bfloat16)

    @pl.when(ph == 2)
    def _():
        gf = _dot_t(hf_sc[rows, :], hf_sc[...])
        lgf = gf * (1.0 / T)
        lg_ref[...] = lgf.astype(jnp.bfloat16)
        riota = jax.lax.broadcasted_iota(jnp.int32, (bm, b), 0) + m * bm
        ciota = jax.lax.broadcasted_iota(jnp.int32, (bm, b), 1)
        offd = riota != ciota
        expl = jnp.where(offd, jnp.exp(lgf), 0.0)
        dlog = jnp.log(jnp.sum(expl, axis=1, keepdims=True))
        dl_ref[...] = jnp.broadcast_to(dlog, (bm, 128))
        sim = gf - jnp.where(riota == ciota, 2.0, 0.0)
        smax = jnp.max(sim, axis=1, keepdims=True)
        y = jnp.min(jnp.where(sim == smax, ciota, b), axis=1, keepdims=True)
        y_ref[...] = jnp.broadcast_to(y, (bm, 128))


def _mm1_kernel(x1_ref, x2_ref, w1_ref, b1_ref, h_ref, s_ref, q_ref):
    hd = pl.program_id(0)
    m = pl.program_id(1)
    xb = jnp.where(hd == 0, x1_ref[...], x2_ref[...]).astype(jnp.bfloat16)
    hf = _dot_t(xb, w1_ref[...]) + b1_ref[...]
    h_ref[...] = hf.astype(jnp.bfloat16)
    ps = jnp.sum(hf, axis=0, keepdims=True)[None]
    pq = jnp.sum(hf * hf, axis=0, keepdims=True)[None]

    @pl.when(m == 0)
    def _():
        s_ref[...] = ps
        q_ref[...] = pq

    @pl.when(m > 0)
    def _():
        s_ref[...] = s_ref[...] + ps
        q_ref[...] = q_ref[...] + pq


def _mm2_kernel(h_ref, s_ref, q_ref, g_ref, be_ref, w2_ref, b2_ref,
                f_ref, *, batch):
    mu = s_ref[0] / batch
    var = q_ref[0] / batch - mu * mu
    scale = g_ref[...] * jax.lax.rsqrt(var + EPS_BN)
    shift = be_ref[...] - mu * scale
    hn = jnp.maximum(h_ref[...].astype(jnp.float32) * scale + shift, 0.0)
    of = _dot_t(hn.astype(jnp.bfloat16), w2_ref[...]) + b2_ref[...]
    ss = jnp.sum(of * of, axis=1, keepdims=True)
    inv = jax.lax.rsqrt(jnp.maximum(ss, 1e-24))
    f_ref[...] = (of * inv).astype(jnp.bfloat16)


def _gram_kernel(fb_ref, ff_ref, lg_ref, y_ref, dl_ref, *, bm, b):
    gf = _dot_t(fb_ref[...], ff_ref[...])
    lgf = gf * (1.0 / T)
    lg_ref[...] = lgf.astype(jnp.bfloat16)
    m = pl.program_id(1)
    riota = jax.lax.broadcasted_iota(jnp.int32, (bm, b), 0) + m * bm
    ciota = jax.lax.broadcasted_iota(jnp.int32, (bm, b), 1)
    offd = riota != ciota
    expl = jnp.where(offd, jnp.exp(lgf), 0.0)
    dlog = jnp.log(jnp.sum(expl, axis=1, keepdims=True))
    dl_ref[...] = jnp.broadcast_to(dlog, (bm, 128))
    sim = gf - jnp.where(riota == ciota, 2.0, 0.0)
    smax = jnp.max(sim, axis=1, keepdims=True)
    y = jnp.min(jnp.where(sim == smax, ciota, b), axis=1, keepdims=True)
    y_ref[...] = jnp.broadcast_to(y, (bm, 128))


def _row_to_col(row8, b):
    slabs = []
    for a in range(b // 128):
        sl = row8[:, 128 * a:128 * (a + 1)]
        rep = jnp.tile(sl, (16, 1))
        slabs.append(jnp.transpose(rep))
    return jnp.concatenate(slabs, axis=0)


def _col_to_row8(col, b):
    outs = []
    for a in range(b // 128):
        t = jnp.transpose(col[128 * a:128 * (a + 1), :])
        outs.append(t[0:8, :])
    return jnp.concatenate(outs, axis=1)


def _min_rows(get_chunk, b):
    acc = jnp.full((8, b), INF, jnp.float32)
    for r0 in range(0, b, 64):
        blk = get_chunk(r0)
        for k in range(8):
            acc = jnp.minimum(acc, blk[8 * k:8 * (k + 1), :])
    return jnp.min(acc, axis=0, keepdims=True)


def _v16_to_col(v16, b):
    slabs = []
    for a in range(b // 128):
        rep = jnp.broadcast_to(v16[a:a + 1, :], (128, 128))
        slabs.append(jnp.transpose(rep))
    return jnp.concatenate(slabs, axis=0)


def _row_to_v16(row, b):
    return jnp.concatenate(
        [row[0:1, 128 * a:128 * (a + 1)] for a in range(b // 128)], axis=0)


def _gather16(v16, r, c, b):
    out = jnp.full((b // 128, 128), INF, jnp.float32)
    for a in range(b // 128):
        rowb = jnp.broadcast_to(v16[a:a + 1, :], (b // 128, 128))
        g_a = jnp.take_along_axis(rowb, c, axis=1)
        out = jnp.where(r == a, g_a, out)
    return out


def _cc_kernel(y_ref, lr_ref, lc_ref, yc_ref, *, b):
    nrep = b // 128
    ycol = y_ref[...]
    ciota = jax.lax.broadcasted_iota(jnp.int32, (b, b), 1)
    yc_ref[...] = jnp.where(jnp.tile(ycol, (1, nrep)) == ciota, 0.0, INF)
    y16 = _row_to_v16(_col_to_row8(ycol, b)[0:1, :], b)
    ry = jax.lax.shift_right_logical(y16, 7)
    cy = jnp.bitwise_and(y16, 127)

    l16 = (jax.lax.broadcasted_iota(jnp.int32, (nrep, 128), 0) * 128 +
           jax.lax.broadcasted_iota(jnp.int32, (nrep, 128), 1)
           ).astype(jnp.float32)

    def body(carry):
        it, _, l16 = carry
        g = _gather16(l16, ry, cy, b)
        ln = jnp.minimum(l16, g)
        l_col = _v16_to_col(l16, b)

        def chunk(r0):
            cl = jnp.tile(l_col[r0:r0 + 64, :], (1, nrep))
            return yc_ref[r0:r0 + 64, :] + cl

        s16 = _row_to_v16(_min_rows(chunk, b), b)
        ln2 = jnp.minimum(ln, s16)
        ln2i = ln2.astype(jnp.int32)
        c16 = _gather16(ln2, jax.lax.shift_right_logical(ln2i, 7),
                        jnp.bitwise_and(ln2i, 127), b)
        l_new = jnp.minimum(ln2, c16)
        changed = jnp.any(l_new != l16)
        return it + 1, changed, l_new

    def cond(carry):
        it, changed, _ = carry
        return jnp.logical_and(it < CC_ITERS, changed)

    _, _, l16 = jax.lax.while_loop(
        cond, body, (jnp.int32(0), jnp.bool_(True), l16))
    row = jnp.concatenate(
        [l16[a:a + 1, :] for a in range(nrep)], axis=1)
    lr_ref[...] = jnp.broadcast_to(row, (8, b)).astype(jnp.int32)[None]
    lc_ref[...] = _v16_to_col(l16, b).astype(jnp.int32)


def _loss_kernel(lg_ref, lc_ref, lr_ref, dl_ref, acc_ref, *, bm, b):
    m = pl.program_id(1)
    lg = lg_ref[...].astype(jnp.float32)
    lab_c = jnp.tile(lc_ref[...], (1, b // 128))
    lab_r = jnp.tile(lr_ref[0], (bm // 8, 1))
    riota = jax.lax.broadcasted_iota(jnp.int32, (bm, b), 0) + m * bm
    ciota = jax.lax.broadcasted_iota(jnp.int32, (bm, b), 1)
    offd = riota != ciota
    mask = jnp.where(jnp.logical_and(lab_c == lab_r, offd), 1.0, 0.0)
    s = jnp.sum(mask * lg, axis=1, keepdims=True)
    cnt = jnp.sum(mask, axis=1, keepdims=True)
    terms = s / cnt - dl_ref[...]
    part = jnp.sum(terms, axis=0, keepdims=True)[None]

    @pl.when(m == 0)
    def _():
        acc_ref[...] = jnp.zeros_like(acc_ref)

    acc_ref[...] = acc_ref[...] + jnp.broadcast_to(part, acc_ref.shape)


def kernel(x1, x2, W1, b1, gamma, beta, W2, b2):
    b, d = x1.shape
    h = W1.shape[0]
    bm = min(512, b)
    mb = b // bm
    bm2 = min(1024, b)
    mb2 = b // bm2
    f32 = jnp.float32
    params = pltpu.CompilerParams(
        dimension_semantics=("parallel", "arbitrary"),
        vmem_limit_bytes=56 * 1024 * 1024)

    ws = jnp.stack([W1, W2]).astype(jnp.bfloat16)
    b1r = b1.reshape(1, h)
    gr = gamma.reshape(1, h)
    ber = beta.reshape(1, h)
    b2r = b2.reshape(1, d)

    params3 = pltpu.CompilerParams(
        dimension_semantics=("parallel", "arbitrary", "arbitrary"),
        vmem_limit_bytes=56 * 1024 * 1024)
    logits, ycol, dlog = pl.pallas_call(
        functools.partial(_head_gram_kernel, bm=bm, b=b, batch=float(b)),
        grid=(2, 3, mb),
        in_specs=[
            pl.BlockSpec(
                (bm, d),
                lambda hd, ph, m: (m * (1 - hd) * ((2 - ph) * (1 - ph) // 2),
                                   0)),
            pl.BlockSpec(
                (bm, d),
                lambda hd, ph, m: (m * hd * ((2 - ph) * (1 - ph) // 2), 0)),
            pl.BlockSpec((1, h, d), lambda hd, ph, m: (ph - ph // 2, 0, 0)),
            pl.BlockSpec((1, h), lambda hd, ph, m: (0, 0)),
            pl.BlockSpec((1, h), lambda hd, ph, m: (0, 0)),
            pl.BlockSpec((1, h), lambda hd, ph, m: (0, 0)),
            pl.BlockSpec((1, d), lambda hd, ph, m: (0, 0)),
        ],
        out_specs=[
            pl.BlockSpec((bm, b),
                         lambda hd, ph, m: (hd * mb + m * (ph // 2), 0)),
            pl.BlockSpec((bm, 128),
                         lambda hd, ph, m: (hd * mb + m * (ph // 2), 0)),
            pl.BlockSpec((bm, 128),
                         lambda hd, ph, m: (hd * mb + m * (ph // 2), 0)),
        ],
        out_shape=[
            jax.ShapeDtypeStruct((2 * b, b), jnp.bfloat16),
            jax.ShapeDtypeStruct((2 * b, 128), jnp.int32),
            jax.ShapeDtypeStruct((2 * b, 128), f32),
        ],
        scratch_shapes=[
            pltpu.VMEM((b, h), jnp.bfloat16),
            pltpu.VMEM((1, h), f32),
            pltpu.VMEM((1, h), f32),
        ],
        compiler_params=params3,
        name="wcl_head_gram",
    )(x1, x2, ws, b1r, gr, ber, b2r)

    labrow, labcol = pl.pallas_call(
        functools.partial(_cc_kernel, b=b),
        grid=(2,),
        in_specs=[pl.BlockSpec((b, 128), lambda hd: (hd, 0))],
        out_specs=[
            pl.BlockSpec((1, 8, b), lambda hd: (hd, 0, 0)),
            pl.BlockSpec((b, 128), lambda hd: (hd, 0)),
        ],
        out_shape=[
            jax.ShapeDtypeStruct((2, 8, b), jnp.int32),
            jax.ShapeDtypeStruct((2 * b, 128), jnp.int32),
        ],
        scratch_shapes=[
            pltpu.VMEM((b, b), f32),
        ],
        compiler_params=pltpu.CompilerParams(
            dimension_semantics=("parallel",),
            vmem_limit_bytes=56 * 1024 * 1024),
        name="wcl_cc",
    )(ycol)

    acc = pl.pallas_call(
        functools.partial(_loss_kernel, bm=bm2, b=b),
        grid=(2, mb2),
        in_specs=[
            pl.BlockSpec((bm2, b), lambda hd, m: (hd * mb2 + m, 0)),
            pl.BlockSpec((bm2, 128), lambda hd, m: ((1 - hd) * mb2 + m, 0)),
            pl.BlockSpec((1, 8, b), lambda hd, m: (1 - hd, 0, 0)),
            pl.BlockSpec((bm2, 128), lambda hd, m: (hd * mb2 + m, 0)),
        ],
        out_specs=pl.BlockSpec((1, 8, 128), lambda hd, m: (hd, 0, 0)),
        out_shape=jax.ShapeDtypeStruct((2, 8, 128), f32),
        compiler_params=params,
        name="wcl_loss",
    )(logits, labcol, labrow, dlog)

    return -(acc[0, 0, 0] + acc[1, 0, 0]) / (2.0 * b)
